# Initial kernel scaffold; baseline (speedup 1.0000x reference)
import jax
import jax.numpy as jnp
from jax.experimental import pallas as pl


def kernel(node_embs, edge_embs, edge_index, W_O, b_O, W_I, b_I, W_S, b_S, bn_gamma, bn_beta):
    raise NotImplementedError("write your pallas kernel here")


# speedup vs baseline: 2.8758x; 2.8758x over previous
import jax
import jax.numpy as jnp
from jax import lax
from jax.experimental import pallas as pl
from jax.experimental.pallas import tpu as pltpu
from jax.experimental.pallas import tpu_sc as plsc

N = 10000
E = 320000
D = 128

NC = 2
NS = 16
K = 80
EPT = E // NS
NCHUNK = EPT // K
GRP = 8
ROWS_MAIN = 624
TAIL = N - NS * ROWS_MAIN


def _sc_aggregate_kernel(x_hbm, sd_hbm, e_hbm, outa_hbm, outb_hbm, outc_hbm,
                         acc_s, eb_v, zb_v, sidx_v, gidx_v, sem0):
    c = lax.axis_index("c")
    s = lax.axis_index("s")
    zvec = jnp.zeros((16,), jnp.float32)

    def init_z(r, _):
        for j in range(D // 16):
            zb_v[r, pl.ds(j * 16, 16)] = zvec
        return 0
    lax.fori_loop(0, GRP, init_z, 0)

    def zero_acc():
        def z(q, _):
            pltpu.sync_copy(zb_v, acc_s.at[pl.ds(s * ROWS_MAIN + q * GRP, GRP)])
            return 0
        lax.fori_loop(0, ROWS_MAIN // GRP, z, 0)

        @pl.when(s == NS - 1)
        def _():
            def z2(q, _):
                pltpu.sync_copy(zb_v, acc_s.at[pl.ds(NS * ROWS_MAIN + q * GRP, GRP)])
                return 0
            lax.fori_loop(0, TAIL // GRP, z2, 0)

    def copy_out(dst_hbm):
        pltpu.sync_copy(acc_s.at[pl.ds(s * ROWS_MAIN, ROWS_MAIN)],
                        dst_hbm.at[c, pl.ds(s * ROWS_MAIN, ROWS_MAIN)])

        @pl.when(s == NS - 1)
        def _():
            pltpu.sync_copy(acc_s.at[pl.ds(NS * ROWS_MAIN, TAIL)],
                            dst_hbm.at[c, pl.ds(NS * ROWS_MAIN, TAIL)])

    ebase = s * EPT
    soff = (1 - c) * E
    goff = c * E

    zero_acc()
    plsc.subcore_barrier()

    def chunk_a(i, _):
        pltpu.sync_copy(sd_hbm.at[pl.ds(soff + ebase + i * K, K)], sidx_v)
        pltpu.sync_copy(e_hbm.at[pl.ds(ebase + i * K, K)], eb_v)
        pltpu.sync_copy(eb_v, acc_s.at[sidx_v], add=True)
        return 0
    lax.fori_loop(0, NCHUNK, chunk_a, 0)

    plsc.subcore_barrier()
    copy_out(outa_hbm)
    zero_acc()
    plsc.subcore_barrier()

    def chunk_b(i, _):
        pltpu.sync_copy(sd_hbm.at[pl.ds(soff + ebase + i * K, K)], sidx_v)
        pltpu.sync_copy(sd_hbm.at[pl.ds(goff + ebase + i * K, K)], gidx_v)
        cp = pltpu.async_copy(x_hbm.at[gidx_v], eb_v, sem0)
        cp.wait()
        pltpu.sync_copy(eb_v, acc_s.at[sidx_v], add=True)
        return 0
    lax.fori_loop(0, NCHUNK, chunk_b, 0)

    plsc.subcore_barrier()
    copy_out(outb_hbm)
    zero_acc()
    plsc.subcore_barrier()

    ovec = jnp.ones((16,), jnp.float32)

    def init_o(r, _):
        for j in range(D // 16):
            eb_v[r, pl.ds(j * 16, 16)] = ovec
        return 0
    lax.fori_loop(0, K, init_o, 0)

    def chunk_c(i, _):
        pltpu.sync_copy(sd_hbm.at[pl.ds(soff + ebase + i * K, K)], sidx_v)
        pltpu.sync_copy(eb_v, acc_s.at[sidx_v], add=True)
        return 0
    lax.fori_loop(0, NCHUNK, chunk_c, 0)

    plsc.subcore_barrier()
    copy_out(outc_hbm)


def _sc_aggregate(node_embs, sd, edge_embs):
    mesh = plsc.VectorSubcoreMesh(core_axis_name="c", subcore_axis_name="s")
    f = pl.kernel(
        _sc_aggregate_kernel,
        out_type=(
            jax.ShapeDtypeStruct((NC, N, D), jnp.float32),
            jax.ShapeDtypeStruct((NC, N, D), jnp.float32),
            jax.ShapeDtypeStruct((NC, N, D), jnp.float32),
        ),
        mesh=mesh,
        scratch_types=[
            pltpu.VMEM_SHARED((N, D), jnp.float32),
            pltpu.VMEM((K, D), jnp.float32),
            pltpu.VMEM((GRP, D), jnp.float32),
            pltpu.VMEM((K,), jnp.int32),
            pltpu.VMEM((K,), jnp.int32),
            pltpu.SemaphoreType.DMA,
        ],
    )
    return f(node_embs, sd, edge_embs)


BLK = 1000
NBLK = N // BLK


def _tc_combine_kernel(x_ref, a0_ref, a1_ref, b0_ref, b1_ref, c0_ref, c1_ref,
                       w_ref, bsum_ref, h_ref, st_ref):
    g = pl.program_id(0)

    @pl.when(g == 0)
    def _():
        st_ref[...] = jnp.zeros_like(st_ref)

    cf = jnp.maximum(c0_ref[...][:, 0:1], 1.0)
    cr = jnp.maximum(c1_ref[...][:, 0:1], 1.0)
    hf = (b0_ref[...] - a0_ref[...]) / cf
    hr = (b1_ref[...] - a1_ref[...]) / cr
    stacked = jnp.concatenate([hf, hr, x_ref[...]], axis=1)
    h = (jnp.dot(stacked, w_ref[...], preferred_element_type=jnp.float32)
         * (1.0 / 3.0) + bsum_ref[...])
    h_ref[...] = h
    st_ref[0:1, :] += jnp.sum(h, axis=0, keepdims=True)
    st_ref[1:2, :] += jnp.sum(h * h, axis=0, keepdims=True)


def _tc_norm_kernel(h_ref, st_ref, gamma_ref, beta_ref, out_ref):
    mean = st_ref[0:1, :] * (1.0 / N)
    var = st_ref[1:2, :] * (1.0 / N) - mean * mean
    scale = gamma_ref[...] * lax.rsqrt(var + 1e-5)
    out_ref[...] = (h_ref[...] - mean) * scale + beta_ref[...]


def _tc_combine(x, a0, a1, b0, b1, c0, c1, w, bsum, gamma, beta):
    row = lambda i: (i, 0)
    h, st = pl.pallas_call(
        _tc_combine_kernel,
        grid=(NBLK,),
        in_specs=[
            pl.BlockSpec((BLK, D), row),
            pl.BlockSpec((BLK, D), row),
            pl.BlockSpec((BLK, D), row),
            pl.BlockSpec((BLK, D), row),
            pl.BlockSpec((BLK, D), row),
            pl.BlockSpec((BLK, D), row),
            pl.BlockSpec((BLK, D), row),
            pl.BlockSpec((3 * D, D), lambda i: (0, 0)),
            pl.BlockSpec((1, D), lambda i: (0, 0)),
        ],
        out_specs=[
            pl.BlockSpec((BLK, D), row),
            pl.BlockSpec((8, D), lambda i: (0, 0)),
        ],
        out_shape=[
            jax.ShapeDtypeStruct((N, D), jnp.float32),
            jax.ShapeDtypeStruct((8, D), jnp.float32),
        ],
    )(x, a0, a1, b0, b1, c0, c1, w, bsum)
    out = pl.pallas_call(
        _tc_norm_kernel,
        grid=(NBLK,),
        in_specs=[
            pl.BlockSpec((BLK, D), row),
            pl.BlockSpec((8, D), lambda i: (0, 0)),
            pl.BlockSpec((1, D), lambda i: (0, 0)),
            pl.BlockSpec((1, D), lambda i: (0, 0)),
        ],
        out_specs=pl.BlockSpec((BLK, D), row),
        out_shape=jax.ShapeDtypeStruct((N, D), jnp.float32),
    )(h, st, gamma, beta)
    return out


@jax.jit
def kernel(node_embs, edge_embs, edge_index, W_O, b_O, W_I, b_I, W_S, b_S,
           bn_gamma, bn_beta):
    sd = edge_index.astype(jnp.int32).reshape(-1)
    outa, outb, outc = _sc_aggregate(node_embs, sd, edge_embs)
    w = jnp.concatenate([W_O.T, W_I.T, W_S.T], axis=0)
    bsum = ((b_O + b_I + b_S) * (1.0 / 3.0)).reshape(1, D)
    return _tc_combine(node_embs, outa[0], outa[1], outb[0], outb[1],
                       outc[0], outc[1], w, bsum,
                       bn_gamma.reshape(1, D), bn_beta.reshape(1, D))


# speedup vs baseline: 5.7130x; 1.9866x over previous
import jax
import jax.numpy as jnp
from jax import lax
from jax.experimental import pallas as pl
from jax.experimental.pallas import tpu as pltpu
from jax.experimental.pallas import tpu_sc as plsc

N = 10000
E = 320000
D = 128

NC = 2
NS = 16
K = 80
EPT = E // NS
NCHUNK = EPT // K
GRP = 8
ROWS_MAIN = 624
TAIL = N - NS * ROWS_MAIN


def _sc_aggregate_kernel(x_hbm, sd_hbm, e_hbm, outa_hbm, outb_hbm, outc_hbm,
                         acc_s, eb0_v, eb1_v, ob_v, zb_v, si0_v, si1_v,
                         gi0_v, gi1_v, se0, se1, ss0, ss1, sg0, sg1):
    c = lax.axis_index("c")
    s = lax.axis_index("s")
    zvec = jnp.zeros((16,), jnp.float32)
    eb = (eb0_v, eb1_v)
    si = (si0_v, si1_v)
    gi = (gi0_v, gi1_v)
    ses = (se0, se1)
    sss = (ss0, ss1)
    sgs = (sg0, sg1)

    def init_z(r, _):
        for j in range(D // 16):
            zb_v[r, pl.ds(j * 16, 16)] = zvec
        return 0
    lax.fori_loop(0, GRP, init_z, 0)

    def zero_acc():
        def z(q, _):
            pltpu.sync_copy(zb_v, acc_s.at[pl.ds(s * ROWS_MAIN + q * GRP, GRP)])
            return 0
        lax.fori_loop(0, ROWS_MAIN // GRP, z, 0)

        @pl.when(s == NS - 1)
        def _():
            def z2(q, _):
                pltpu.sync_copy(zb_v, acc_s.at[pl.ds(NS * ROWS_MAIN + q * GRP, GRP)])
                return 0
            lax.fori_loop(0, TAIL // GRP, z2, 0)

    def copy_out(dst_hbm):
        pltpu.sync_copy(acc_s.at[pl.ds(s * ROWS_MAIN, ROWS_MAIN)],
                        dst_hbm.at[c, pl.ds(s * ROWS_MAIN, ROWS_MAIN)])

        @pl.when(s == NS - 1)
        def _():
            pltpu.sync_copy(acc_s.at[pl.ds(NS * ROWS_MAIN, TAIL)],
                            dst_hbm.at[c, pl.ds(NS * ROWS_MAIN, TAIL)])

    ovec = jnp.ones((16,), jnp.float32)

    def init_o(r, _):
        for j in range(D // 16):
            ob_v[r, pl.ds(j * 16, 16)] = ovec
        return 0
    lax.fori_loop(0, K, init_o, 0)

    ebase = s * EPT
    soff = (1 - c) * E
    goff = c * E

    def sidx_issue(j, p):
        pltpu.async_copy(sd_hbm.at[pl.ds(soff + ebase + j * K, K)], si[p], sss[p])

    def sidx_wait(j, p):
        pltpu.make_async_copy(
            sd_hbm.at[pl.ds(soff + ebase + j * K, K)], si[p], sss[p]).wait()

    def gidx_issue(j, p):
        pltpu.async_copy(sd_hbm.at[pl.ds(goff + ebase + j * K, K)], gi[p], sgs[p])

    def gidx_wait(j, p):
        pltpu.make_async_copy(
            sd_hbm.at[pl.ds(goff + ebase + j * K, K)], gi[p], sgs[p]).wait()

    def edge_issue(j, p):
        pltpu.async_copy(e_hbm.at[pl.ds(ebase + j * K, K)], eb[p], ses[p])

    def edge_wait(j, p):
        pltpu.make_async_copy(
            e_hbm.at[pl.ds(ebase + j * K, K)], eb[p], ses[p]).wait()

    def gather_issue(p):
        pltpu.async_copy(x_hbm.at[gi[p]], eb[p], ses[p])

    def gather_wait(p):
        pltpu.make_async_copy(x_hbm.at[gi[p]], eb[p], ses[p]).wait()

    HALF = NCHUNK // 2

    zero_acc()
    plsc.subcore_barrier()

    sidx_issue(0, 0)
    edge_issue(0, 0)
    sidx_issue(1, 1)
    edge_issue(1, 1)

    def outer_a(i2, _):
        for p in (0, 1):
            j = i2 * 2 + p
            edge_wait(j, p)
            sidx_wait(j, p)
            pltpu.sync_copy(eb[p], acc_s.at[si[p]], add=True)

            @pl.when(i2 < HALF - 1)
            def _():
                sidx_issue(j + 2, p)
                edge_issue(j + 2, p)
        return 0
    lax.fori_loop(0, HALF, outer_a, 0)

    plsc.subcore_barrier()
    copy_out(outa_hbm)
    zero_acc()
    plsc.subcore_barrier()

    sidx_issue(0, 0)
    gidx_issue(0, 0)
    gidx_wait(0, 0)
    gather_issue(0)
    sidx_issue(1, 1)
    gidx_issue(1, 1)

    def outer_b(i2, _):
        for p in (0, 1):
            j = i2 * 2 + p
            gather_wait(p)
            if p == 0:
                gidx_wait(j + 1, 1)
                gather_issue(1)
            else:
                @pl.when(i2 < HALF - 1)
                def _():
                    gidx_wait(j + 1, 0)
                    gather_issue(0)
            sidx_wait(j, p)
            pltpu.sync_copy(eb[p], acc_s.at[si[p]], add=True)

            @pl.when(i2 < HALF - 1)
            def _():
                sidx_issue(j + 2, p)
                gidx_issue(j + 2, p)
        return 0
    lax.fori_loop(0, HALF, outer_b, 0)

    plsc.subcore_barrier()
    copy_out(outb_hbm)
    zero_acc()
    plsc.subcore_barrier()

    sidx_issue(0, 0)
    sidx_issue(1, 1)

    def outer_c(i2, _):
        for p in (0, 1):
            j = i2 * 2 + p
            sidx_wait(j, p)
            pltpu.sync_copy(ob_v, acc_s.at[si[p]], add=True)

            @pl.when(i2 < HALF - 1)
            def _():
                sidx_issue(j + 2, p)
        return 0
    lax.fori_loop(0, HALF, outer_c, 0)

    plsc.subcore_barrier()
    copy_out(outc_hbm)


def _sc_aggregate(node_embs, sd, edge_embs):
    mesh = plsc.VectorSubcoreMesh(core_axis_name="c", subcore_axis_name="s")
    f = pl.kernel(
        _sc_aggregate_kernel,
        out_type=(
            jax.ShapeDtypeStruct((NC, N, D), jnp.float32),
            jax.ShapeDtypeStruct((NC, N, D), jnp.float32),
            jax.ShapeDtypeStruct((NC, N, D), jnp.float32),
        ),
        mesh=mesh,
        scratch_types=[
            pltpu.VMEM_SHARED((N, D), jnp.float32),
            pltpu.VMEM((K, D), jnp.float32),
            pltpu.VMEM((K, D), jnp.float32),
            pltpu.VMEM((K, D), jnp.float32),
            pltpu.VMEM((GRP, D), jnp.float32),
            pltpu.VMEM((K,), jnp.int32),
            pltpu.VMEM((K,), jnp.int32),
            pltpu.VMEM((K,), jnp.int32),
            pltpu.VMEM((K,), jnp.int32),
            pltpu.SemaphoreType.DMA,
            pltpu.SemaphoreType.DMA,
            pltpu.SemaphoreType.DMA,
            pltpu.SemaphoreType.DMA,
            pltpu.SemaphoreType.DMA,
            pltpu.SemaphoreType.DMA,
        ],
    )
    return f(node_embs, sd, edge_embs)


BLK = 1000
NBLK = N // BLK


def _tc_combine_kernel(x_ref, a0_ref, a1_ref, b0_ref, b1_ref, c0_ref, c1_ref,
                       w_ref, bsum_ref, h_ref, st_ref):
    g = pl.program_id(0)

    @pl.when(g == 0)
    def _():
        st_ref[...] = jnp.zeros_like(st_ref)

    cf = jnp.maximum(c0_ref[...][:, 0:1], 1.0)
    cr = jnp.maximum(c1_ref[...][:, 0:1], 1.0)
    hf = (b0_ref[...] - a0_ref[...]) / cf
    hr = (b1_ref[...] - a1_ref[...]) / cr
    stacked = jnp.concatenate([hf, hr, x_ref[...]], axis=1)
    h = (jnp.dot(stacked, w_ref[...], preferred_element_type=jnp.float32)
         * (1.0 / 3.0) + bsum_ref[...])
    h_ref[...] = h
    st_ref[0:1, :] += jnp.sum(h, axis=0, keepdims=True)
    st_ref[1:2, :] += jnp.sum(h * h, axis=0, keepdims=True)


def _tc_norm_kernel(h_ref, st_ref, gamma_ref, beta_ref, out_ref):
    mean = st_ref[0:1, :] * (1.0 / N)
    var = st_ref[1:2, :] * (1.0 / N) - mean * mean
    scale = gamma_ref[...] * lax.rsqrt(var + 1e-5)
    out_ref[...] = (h_ref[...] - mean) * scale + beta_ref[...]


def _tc_combine(x, a0, a1, b0, b1, c0, c1, w, bsum, gamma, beta):
    row = lambda i: (i, 0)
    h, st = pl.pallas_call(
        _tc_combine_kernel,
        grid=(NBLK,),
        in_specs=[
            pl.BlockSpec((BLK, D), row),
            pl.BlockSpec((BLK, D), row),
            pl.BlockSpec((BLK, D), row),
            pl.BlockSpec((BLK, D), row),
            pl.BlockSpec((BLK, D), row),
            pl.BlockSpec((BLK, D), row),
            pl.BlockSpec((BLK, D), row),
            pl.BlockSpec((3 * D, D), lambda i: (0, 0)),
            pl.BlockSpec((1, D), lambda i: (0, 0)),
        ],
        out_specs=[
            pl.BlockSpec((BLK, D), row),
            pl.BlockSpec((8, D), lambda i: (0, 0)),
        ],
        out_shape=[
            jax.ShapeDtypeStruct((N, D), jnp.float32),
            jax.ShapeDtypeStruct((8, D), jnp.float32),
        ],
    )(x, a0, a1, b0, b1, c0, c1, w, bsum)
    out = pl.pallas_call(
        _tc_norm_kernel,
        grid=(NBLK,),
        in_specs=[
            pl.BlockSpec((BLK, D), row),
            pl.BlockSpec((8, D), lambda i: (0, 0)),
            pl.BlockSpec((1, D), lambda i: (0, 0)),
            pl.BlockSpec((1, D), lambda i: (0, 0)),
        ],
        out_specs=pl.BlockSpec((BLK, D), row),
        out_shape=jax.ShapeDtypeStruct((N, D), jnp.float32),
    )(h, st, gamma, beta)
    return out


@jax.jit
def kernel(node_embs, edge_embs, edge_index, W_O, b_O, W_I, b_I, W_S, b_S,
           bn_gamma, bn_beta):
    sd = edge_index.astype(jnp.int32).reshape(-1)
    outa, outb, outc = _sc_aggregate(node_embs, sd, edge_embs)
    w = jnp.concatenate([W_O.T, W_I.T, W_S.T], axis=0)
    bsum = ((b_O + b_I + b_S) * (1.0 / 3.0)).reshape(1, D)
    return _tc_combine(node_embs, outa[0], outa[1], outb[0], outb[1],
                       outc[0], outc[1], w, bsum,
                       bn_gamma.reshape(1, D), bn_beta.reshape(1, D))


# speedup vs baseline: 5.8428x; 1.0227x over previous
import jax
import jax.numpy as jnp
from jax import lax
from jax.experimental import pallas as pl
from jax.experimental.pallas import tpu as pltpu
from jax.experimental.pallas import tpu_sc as plsc

N = 10000
E = 320000
D = 128

NC = 2
NS = 16
K = 80
EPT = E // NS
NCHUNK = EPT // K
GRP = 8
ROWS_MAIN = 624
TAIL = N - NS * ROWS_MAIN


def _sc_aggregate_kernel(x_hbm, sd_hbm, e_hbm, outa_hbm, outb_hbm, outc_hbm,
                         acc_s, eb0_v, eb1_v, ob_v, zb_v, si0_v, si1_v,
                         gi0_v, gi1_v, se0, se1, ss0, ss1, sg0, sg1, sc0, sc1):
    c = lax.axis_index("c")
    s = lax.axis_index("s")
    zvec = jnp.zeros((16,), jnp.float32)
    eb = (eb0_v, eb1_v)
    si = (si0_v, si1_v)
    gi = (gi0_v, gi1_v)
    ses = (se0, se1)
    sss = (ss0, ss1)
    sgs = (sg0, sg1)

    def init_z(r, _):
        for j in range(D // 16):
            zb_v[r, pl.ds(j * 16, 16)] = zvec
        return 0
    lax.fori_loop(0, K, init_z, 0)

    def zero_acc():
        for q in range(ROWS_MAIN // K):
            pltpu.sync_copy(zb_v, acc_s.at[pl.ds(s * ROWS_MAIN + q * K, K)])
        rem = ROWS_MAIN % K
        pltpu.sync_copy(zb_v.at[pl.ds(0, rem)],
                        acc_s.at[pl.ds(s * ROWS_MAIN + (ROWS_MAIN // K) * K, rem)])

        @pl.when(s == NS - 1)
        def _():
            pltpu.sync_copy(zb_v.at[pl.ds(0, TAIL)],
                            acc_s.at[pl.ds(NS * ROWS_MAIN, TAIL)])

    def copy_out(dst_hbm):
        pltpu.sync_copy(acc_s.at[pl.ds(s * ROWS_MAIN, ROWS_MAIN)],
                        dst_hbm.at[c, pl.ds(s * ROWS_MAIN, ROWS_MAIN)])

        @pl.when(s == NS - 1)
        def _():
            pltpu.sync_copy(acc_s.at[pl.ds(NS * ROWS_MAIN, TAIL)],
                            dst_hbm.at[c, pl.ds(NS * ROWS_MAIN, TAIL)])

    ovec = jnp.ones((16,), jnp.float32)

    def init_o(r, _):
        for j in range(D // 16):
            ob_v[r, pl.ds(j * 16, 16)] = ovec
        return 0
    lax.fori_loop(0, K, init_o, 0)

    ebase = s * EPT
    soff = (1 - c) * E
    goff = c * E

    def sidx_issue(j, p):
        pltpu.async_copy(sd_hbm.at[pl.ds(soff + ebase + j * K, K)], si[p], sss[p])

    def sidx_wait(j, p):
        pltpu.make_async_copy(
            sd_hbm.at[pl.ds(soff + ebase + j * K, K)], si[p], sss[p]).wait()

    def gidx_issue(j, p):
        pltpu.async_copy(sd_hbm.at[pl.ds(goff + ebase + j * K, K)], gi[p], sgs[p])

    def gidx_wait(j, p):
        pltpu.make_async_copy(
            sd_hbm.at[pl.ds(goff + ebase + j * K, K)], gi[p], sgs[p]).wait()

    def edge_issue(j, p):
        pltpu.async_copy(e_hbm.at[pl.ds(ebase + j * K, K)], eb[p], ses[p])

    def edge_wait(j, p):
        pltpu.make_async_copy(
            e_hbm.at[pl.ds(ebase + j * K, K)], eb[p], ses[p]).wait()

    def gather_issue(p):
        pltpu.async_copy(x_hbm.at[gi[p]], eb[p], ses[p])

    def gather_wait(p):
        pltpu.make_async_copy(x_hbm.at[gi[p]], eb[p], ses[p]).wait()

    HALF = NCHUNK // 2

    zero_acc()
    plsc.subcore_barrier()

    sidx_issue(0, 0)
    edge_issue(0, 0)
    sidx_issue(1, 1)
    edge_issue(1, 1)

    def outer_a(i2, _):
        for p in (0, 1):
            j = i2 * 2 + p
            edge_wait(j, p)
            sidx_wait(j, p)
            pltpu.sync_copy(eb[p], acc_s.at[si[p]], add=True)

            @pl.when(i2 < HALF - 1)
            def _():
                sidx_issue(j + 2, p)
                edge_issue(j + 2, p)
        return 0
    lax.fori_loop(0, HALF, outer_a, 0)

    plsc.subcore_barrier()
    copy_out(outa_hbm)
    zero_acc()
    plsc.subcore_barrier()

    sidx_issue(0, 0)
    gidx_issue(0, 0)
    gidx_wait(0, 0)
    gather_issue(0)
    sidx_issue(1, 1)
    gidx_issue(1, 1)

    def outer_b(i2, _):
        for p in (0, 1):
            j = i2 * 2 + p
            gather_wait(p)
            if p == 0:
                gidx_wait(j + 1, 1)
                gather_issue(1)
            else:
                @pl.when(i2 < HALF - 1)
                def _():
                    gidx_wait(j + 1, 0)
                    gather_issue(0)
            sidx_wait(j, p)
            pltpu.sync_copy(eb[p], acc_s.at[si[p]], add=True)

            @pl.when(i2 < HALF - 1)
            def _():
                sidx_issue(j + 2, p)
                gidx_issue(j + 2, p)
        return 0
    lax.fori_loop(0, HALF, outer_b, 0)

    plsc.subcore_barrier()
    copy_out(outb_hbm)
    zero_acc()
    plsc.subcore_barrier()

    for jr in (NCHUNK - 2, NCHUNK - 1):
        sidx_issue(jr, 0)
        sidx_wait(jr, 0)
        pltpu.sync_copy(ob_v, acc_s.at[si[0]], add=True)

    si4 = (si0_v, si1_v, gi0_v, gi1_v)
    ssi4 = (ss0, ss1, sg0, sg1)
    ssc4 = (se0, se1, sc0, sc1)

    def c_idx_issue(j, q):
        pltpu.async_copy(sd_hbm.at[pl.ds(soff + ebase + j * K, K)],
                         si4[q], ssi4[q])

    def c_idx_wait(j, q):
        pltpu.make_async_copy(sd_hbm.at[pl.ds(soff + ebase + j * K, K)],
                              si4[q], ssi4[q]).wait()

    def c_sc_issue(q):
        pltpu.async_copy(ob_v, acc_s.at[si4[q]], ssc4[q], add=True)

    def c_sc_wait(q):
        pltpu.make_async_copy(ob_v, acc_s.at[si4[q]], ssc4[q]).wait()

    c_idx_issue(0, 0)
    c_idx_issue(1, 1)
    NC4 = (NCHUNK - 2) // 4

    def outer_c(i4, _):
        for q in range(4):
            j = i4 * 4 + q
            c_idx_wait(j, q)
            c_sc_issue(q)
            qn = (q + 2) % 4
            if q < 2:
                @pl.when(i4 > 0)
                def _():
                    c_sc_wait(qn)
                c_idx_issue(j + 2, qn)
            else:
                c_sc_wait(qn)

                @pl.when(i4 < NC4 - 1)
                def _():
                    c_idx_issue(j + 2, qn)
        return 0
    lax.fori_loop(0, NC4, outer_c, 0)
    c_sc_wait(2)
    c_sc_wait(3)

    plsc.subcore_barrier()
    copy_out(outc_hbm)


def _sc_aggregate(node_embs, sd, edge_embs):
    mesh = plsc.VectorSubcoreMesh(core_axis_name="c", subcore_axis_name="s")
    f = pl.kernel(
        _sc_aggregate_kernel,
        out_type=(
            jax.ShapeDtypeStruct((NC, N, D), jnp.float32),
            jax.ShapeDtypeStruct((NC, N, D), jnp.float32),
            jax.ShapeDtypeStruct((NC, N, D), jnp.float32),
        ),
        mesh=mesh,
        scratch_types=[
            pltpu.VMEM_SHARED((N, D), jnp.float32),
            pltpu.VMEM((K, D), jnp.float32),
            pltpu.VMEM((K, D), jnp.float32),
            pltpu.VMEM((K, D), jnp.float32),
            pltpu.VMEM((K, D), jnp.float32),
            pltpu.VMEM((K,), jnp.int32),
            pltpu.VMEM((K,), jnp.int32),
            pltpu.VMEM((K,), jnp.int32),
            pltpu.VMEM((K,), jnp.int32),
            pltpu.SemaphoreType.DMA,
            pltpu.SemaphoreType.DMA,
            pltpu.SemaphoreType.DMA,
            pltpu.SemaphoreType.DMA,
            pltpu.SemaphoreType.DMA,
            pltpu.SemaphoreType.DMA,
            pltpu.SemaphoreType.DMA,
            pltpu.SemaphoreType.DMA,
        ],
    )
    return f(node_embs, sd, edge_embs)


BLK = 1000
NBLK = N // BLK


def _tc_combine_kernel(x_ref, a0_ref, a1_ref, b0_ref, b1_ref, c0_ref, c1_ref,
                       w_ref, bsum_ref, h_ref, st_ref):
    g = pl.program_id(0)

    @pl.when(g == 0)
    def _():
        st_ref[...] = jnp.zeros_like(st_ref)

    cf = jnp.maximum(c0_ref[...][:, 0:1], 1.0)
    cr = jnp.maximum(c1_ref[...][:, 0:1], 1.0)
    hf = (b0_ref[...] - a0_ref[...]) / cf
    hr = (b1_ref[...] - a1_ref[...]) / cr
    stacked = jnp.concatenate([hf, hr, x_ref[...]], axis=1)
    h = (jnp.dot(stacked, w_ref[...], preferred_element_type=jnp.float32)
         * (1.0 / 3.0) + bsum_ref[...])
    h_ref[...] = h
    st_ref[0:1, :] += jnp.sum(h, axis=0, keepdims=True)
    st_ref[1:2, :] += jnp.sum(h * h, axis=0, keepdims=True)


def _tc_norm_kernel(h_ref, st_ref, gamma_ref, beta_ref, out_ref):
    mean = st_ref[0:1, :] * (1.0 / N)
    var = st_ref[1:2, :] * (1.0 / N) - mean * mean
    scale = gamma_ref[...] * lax.rsqrt(var + 1e-5)
    out_ref[...] = (h_ref[...] - mean) * scale + beta_ref[...]


def _tc_combine(x, a0, a1, b0, b1, c0, c1, w, bsum, gamma, beta):
    row = lambda i: (i, 0)
    h, st = pl.pallas_call(
        _tc_combine_kernel,
        grid=(NBLK,),
        in_specs=[
            pl.BlockSpec((BLK, D), row),
            pl.BlockSpec((BLK, D), row),
            pl.BlockSpec((BLK, D), row),
            pl.BlockSpec((BLK, D), row),
            pl.BlockSpec((BLK, D), row),
            pl.BlockSpec((BLK, D), row),
            pl.BlockSpec((BLK, D), row),
            pl.BlockSpec((3 * D, D), lambda i: (0, 0)),
            pl.BlockSpec((1, D), lambda i: (0, 0)),
        ],
        out_specs=[
            pl.BlockSpec((BLK, D), row),
            pl.BlockSpec((8, D), lambda i: (0, 0)),
        ],
        out_shape=[
            jax.ShapeDtypeStruct((N, D), jnp.float32),
            jax.ShapeDtypeStruct((8, D), jnp.float32),
        ],
    )(x, a0, a1, b0, b1, c0, c1, w, bsum)
    out = pl.pallas_call(
        _tc_norm_kernel,
        grid=(NBLK,),
        in_specs=[
            pl.BlockSpec((BLK, D), row),
            pl.BlockSpec((8, D), lambda i: (0, 0)),
            pl.BlockSpec((1, D), lambda i: (0, 0)),
            pl.BlockSpec((1, D), lambda i: (0, 0)),
        ],
        out_specs=pl.BlockSpec((BLK, D), row),
        out_shape=jax.ShapeDtypeStruct((N, D), jnp.float32),
    )(h, st, gamma, beta)
    return out


@jax.jit
def kernel(node_embs, edge_embs, edge_index, W_O, b_O, W_I, b_I, W_S, b_S,
           bn_gamma, bn_beta):
    sd = edge_index.astype(jnp.int32).reshape(-1)
    outa, outb, outc = _sc_aggregate(node_embs, sd, edge_embs)
    w = jnp.concatenate([W_O.T, W_I.T, W_S.T], axis=0)
    bsum = ((b_O + b_I + b_S) * (1.0 / 3.0)).reshape(1, D)
    return _tc_combine(node_embs, outa[0], outa[1], outb[0], outb[1],
                       outc[0], outc[1], w, bsum,
                       bn_gamma.reshape(1, D), bn_beta.reshape(1, D))


# speedup vs baseline: 5.8468x; 1.0007x over previous
import jax
import jax.numpy as jnp
from jax import lax
from jax.experimental import pallas as pl
from jax.experimental.pallas import tpu as pltpu
from jax.experimental.pallas import tpu_sc as plsc

N = 10000
E = 320000
D = 128

NC = 2
NS = 16
K = 80
EPT = E // NS
NCHUNK = EPT // K
ROWS_MAIN = 624
TAIL = N - NS * ROWS_MAIN


def _sc_aggregate_kernel(x_hbm, sd_hbm, e_hbm, outa_hbm, outb_hbm, outc_hbm,
                         acc_s, eb0_v, eb1_v, ob_v, zb_v, si0_v, si1_v,
                         gi0_v, gi1_v, se0, se1, ss0, ss1, sg0, sg1, sc0, sc1):
    c = lax.axis_index("c")
    s = lax.axis_index("s")
    zvec = jnp.zeros((16,), jnp.float32)
    eb = (eb0_v, eb1_v)
    si = (si0_v, si1_v)
    gi = (gi0_v, gi1_v)
    ses = (se0, se1)
    sss = (ss0, ss1)
    sgs = (sg0, sg1)

    def init_z(r, _):
        for j in range(D // 16):
            zb_v[r, pl.ds(j * 16, 16)] = zvec
        return 0
    lax.fori_loop(0, K, init_z, 0)

    def zero_acc():
        for q in range(ROWS_MAIN // K):
            pltpu.sync_copy(zb_v, acc_s.at[pl.ds(s * ROWS_MAIN + q * K, K)])
        rem = ROWS_MAIN % K
        pltpu.sync_copy(zb_v.at[pl.ds(0, rem)],
                        acc_s.at[pl.ds(s * ROWS_MAIN + (ROWS_MAIN // K) * K, rem)])

        @pl.when(s == NS - 1)
        def _():
            pltpu.sync_copy(zb_v.at[pl.ds(0, TAIL)],
                            acc_s.at[pl.ds(NS * ROWS_MAIN, TAIL)])

    def copy_out(dst_hbm):
        pltpu.sync_copy(acc_s.at[pl.ds(s * ROWS_MAIN, ROWS_MAIN)],
                        dst_hbm.at[c, pl.ds(s * ROWS_MAIN, ROWS_MAIN)])

        @pl.when(s == NS - 1)
        def _():
            pltpu.sync_copy(acc_s.at[pl.ds(NS * ROWS_MAIN, TAIL)],
                            dst_hbm.at[c, pl.ds(NS * ROWS_MAIN, TAIL)])

    ovec = jnp.ones((16,), jnp.float32)

    def init_o(r, _):
        for j in range(D // 16):
            ob_v[r, pl.ds(j * 16, 16)] = ovec
        return 0
    lax.fori_loop(0, K, init_o, 0)

    ebase = s * EPT
    soff = (1 - c) * E
    goff = c * E

    def sidx_issue(j, p):
        pltpu.async_copy(sd_hbm.at[pl.ds(soff + ebase + j * K, K)], si[p], sss[p])

    def sidx_wait(j, p):
        pltpu.make_async_copy(
            sd_hbm.at[pl.ds(soff + ebase + j * K, K)], si[p], sss[p]).wait()

    def gidx_issue(j, p):
        pltpu.async_copy(sd_hbm.at[pl.ds(goff + ebase + j * K, K)], gi[p], sgs[p])

    def gidx_wait(j, p):
        pltpu.make_async_copy(
            sd_hbm.at[pl.ds(goff + ebase + j * K, K)], gi[p], sgs[p]).wait()

    def edge_issue(j, p):
        pltpu.async_copy(e_hbm.at[pl.ds(ebase + j * K, K)], eb[p], ses[p])

    def edge_wait(j, p):
        pltpu.make_async_copy(
            e_hbm.at[pl.ds(ebase + j * K, K)], eb[p], ses[p]).wait()

    def gather_issue(p):
        pltpu.async_copy(x_hbm.at[gi[p]], eb[p], ses[p])

    def gather_wait(p):
        pltpu.make_async_copy(x_hbm.at[gi[p]], eb[p], ses[p]).wait()

    HALF = NCHUNK // 2

    zero_acc()
    plsc.subcore_barrier()

    sidx_issue(0, 0)
    edge_issue(0, 0)
    sidx_issue(1, 1)
    edge_issue(1, 1)

    def outer_a(i2, _):
        for p in (0, 1):
            j = i2 * 2 + p
            edge_wait(j, p)
            sidx_wait(j, p)
            pltpu.sync_copy(eb[p], acc_s.at[si[p]], add=True)

            @pl.when(i2 < HALF - 1)
            def _():
                sidx_issue(j + 2, p)
                edge_issue(j + 2, p)
        return 0
    lax.fori_loop(0, HALF, outer_a, 0)

    plsc.subcore_barrier()
    copy_out(outa_hbm)
    zero_acc()
    plsc.subcore_barrier()

    sidx_issue(0, 0)
    gidx_issue(0, 0)
    gidx_wait(0, 0)
    gather_issue(0)
    sidx_issue(1, 1)
    gidx_issue(1, 1)

    def outer_b(i2, _):
        for p in (0, 1):
            j = i2 * 2 + p
            gather_wait(p)
            if p == 0:
                gidx_wait(j + 1, 1)
                gather_issue(1)
            else:
                @pl.when(i2 < HALF - 1)
                def _():
                    gidx_wait(j + 1, 0)
                    gather_issue(0)
            sidx_wait(j, p)
            pltpu.sync_copy(eb[p], acc_s.at[si[p]], add=True)

            @pl.when(i2 < HALF - 1)
            def _():
                sidx_issue(j + 2, p)
                gidx_issue(j + 2, p)
        return 0
    lax.fori_loop(0, HALF, outer_b, 0)

    plsc.subcore_barrier()
    copy_out(outb_hbm)
    zero_acc()
    plsc.subcore_barrier()

    for jr in (NCHUNK - 2, NCHUNK - 1):
        sidx_issue(jr, 0)
        sidx_wait(jr, 0)
        pltpu.sync_copy(ob_v, acc_s.at[si[0]], add=True)

    si4 = (si0_v, si1_v, gi0_v, gi1_v)
    ssi4 = (ss0, ss1, sg0, sg1)
    ssc4 = (se0, se1, sc0, sc1)

    def c_idx_issue(j, q):
        pltpu.async_copy(sd_hbm.at[pl.ds(soff + ebase + j * K, K)],
                         si4[q], ssi4[q])

    def c_idx_wait(j, q):
        pltpu.make_async_copy(sd_hbm.at[pl.ds(soff + ebase + j * K, K)],
                              si4[q], ssi4[q]).wait()

    def c_sc_issue(q):
        pltpu.async_copy(ob_v, acc_s.at[si4[q]], ssc4[q], add=True)

    def c_sc_wait(q):
        pltpu.make_async_copy(ob_v, acc_s.at[si4[q]], ssc4[q]).wait()

    c_idx_issue(0, 0)
    c_idx_issue(1, 1)
    NC4 = (NCHUNK - 2) // 4

    def outer_c(i4, _):
        for q in range(4):
            j = i4 * 4 + q
            c_idx_wait(j, q)
            c_sc_issue(q)
            qn = (q + 2) % 4
            if q < 2:
                @pl.when(i4 > 0)
                def _():
                    c_sc_wait(qn)
                c_idx_issue(j + 2, qn)
            else:
                c_sc_wait(qn)

                @pl.when(i4 < NC4 - 1)
                def _():
                    c_idx_issue(j + 2, qn)
        return 0
    lax.fori_loop(0, NC4, outer_c, 0)
    c_sc_wait(2)
    c_sc_wait(3)

    plsc.subcore_barrier()
    copy_out(outc_hbm)


def _sc_aggregate(node_embs, sd, edge_embs):
    mesh = plsc.VectorSubcoreMesh(core_axis_name="c", subcore_axis_name="s")
    f = pl.kernel(
        _sc_aggregate_kernel,
        out_type=(
            jax.ShapeDtypeStruct((NC, N, D), jnp.float32),
            jax.ShapeDtypeStruct((NC, N, D), jnp.float32),
            jax.ShapeDtypeStruct((NC, N, D), jnp.float32),
        ),
        mesh=mesh,
        scratch_types=[
            pltpu.VMEM_SHARED((N, D), jnp.float32),
            pltpu.VMEM((K, D), jnp.float32),
            pltpu.VMEM((K, D), jnp.float32),
            pltpu.VMEM((K, D), jnp.float32),
            pltpu.VMEM((K, D), jnp.float32),
            pltpu.VMEM((K,), jnp.int32),
            pltpu.VMEM((K,), jnp.int32),
            pltpu.VMEM((K,), jnp.int32),
            pltpu.VMEM((K,), jnp.int32),
            pltpu.SemaphoreType.DMA,
            pltpu.SemaphoreType.DMA,
            pltpu.SemaphoreType.DMA,
            pltpu.SemaphoreType.DMA,
            pltpu.SemaphoreType.DMA,
            pltpu.SemaphoreType.DMA,
            pltpu.SemaphoreType.DMA,
            pltpu.SemaphoreType.DMA,
        ],
    )
    return f(node_embs, sd, edge_embs)


BLK = 1000
NBLK = N // BLK


def _tc_combine_kernel(x_ref, a0_ref, a1_ref, b0_ref, b1_ref, c0_ref, c1_ref,
                       w_ref, bsum_ref, h_ref, st_ref):
    g = pl.program_id(0)

    @pl.when(g == 0)
    def _():
        st_ref[...] = jnp.zeros_like(st_ref)

    cf = jnp.maximum(c0_ref[...][:, 0:1], 1.0)
    cr = jnp.maximum(c1_ref[...][:, 0:1], 1.0)
    hf = (b0_ref[...] - a0_ref[...]) / cf
    hr = (b1_ref[...] - a1_ref[...]) / cr
    stacked = jnp.concatenate([hf, hr, x_ref[...]], axis=1)
    h = (jnp.dot(stacked, w_ref[...], preferred_element_type=jnp.float32)
         * (1.0 / 3.0) + bsum_ref[...])
    h_ref[...] = h
    st_ref[0:1, :] += jnp.sum(h, axis=0, keepdims=True)
    st_ref[1:2, :] += jnp.sum(h * h, axis=0, keepdims=True)


def _tc_norm_kernel(h_ref, st_ref, gamma_ref, beta_ref, out_ref):
    mean = st_ref[0:1, :] * (1.0 / N)
    var = st_ref[1:2, :] * (1.0 / N) - mean * mean
    scale = gamma_ref[...] * lax.rsqrt(var + 1e-5)
    out_ref[...] = (h_ref[...] - mean) * scale + beta_ref[...]


def _tc_combine(x, a0, a1, b0, b1, c0, c1, w, bsum, gamma, beta):
    row = lambda i: (i, 0)
    h, st = pl.pallas_call(
        _tc_combine_kernel,
        grid=(NBLK,),
        in_specs=[
            pl.BlockSpec((BLK, D), row),
            pl.BlockSpec((BLK, D), row),
            pl.BlockSpec((BLK, D), row),
            pl.BlockSpec((BLK, D), row),
            pl.BlockSpec((BLK, D), row),
            pl.BlockSpec((BLK, D), row),
            pl.BlockSpec((BLK, D), row),
            pl.BlockSpec((3 * D, D), lambda i: (0, 0)),
            pl.BlockSpec((1, D), lambda i: (0, 0)),
        ],
        out_specs=[
            pl.BlockSpec((BLK, D), row),
            pl.BlockSpec((8, D), lambda i: (0, 0)),
        ],
        out_shape=[
            jax.ShapeDtypeStruct((N, D), jnp.float32),
            jax.ShapeDtypeStruct((8, D), jnp.float32),
        ],
    )(x, a0, a1, b0, b1, c0, c1, w, bsum)
    out = pl.pallas_call(
        _tc_norm_kernel,
        grid=(NBLK,),
        in_specs=[
            pl.BlockSpec((BLK, D), row),
            pl.BlockSpec((8, D), lambda i: (0, 0)),
            pl.BlockSpec((1, D), lambda i: (0, 0)),
            pl.BlockSpec((1, D), lambda i: (0, 0)),
        ],
        out_specs=pl.BlockSpec((BLK, D), row),
        out_shape=jax.ShapeDtypeStruct((N, D), jnp.float32),
    )(h, st, gamma, beta)
    return out


@jax.jit
def kernel(node_embs, edge_embs, edge_index, W_O, b_O, W_I, b_I, W_S, b_S,
           bn_gamma, bn_beta):
    sd = edge_index.astype(jnp.int32).reshape(-1)
    outa, outb, outc = _sc_aggregate(node_embs, sd, edge_embs)
    w = jnp.concatenate([W_O.T, W_I.T, W_S.T], axis=0)
    bsum = ((b_O + b_I + b_S) * (1.0 / 3.0)).reshape(1, D)
    return _tc_combine(node_embs, outa[0], outa[1], outb[0], outb[1],
                       outc[0], outc[1], w, bsum,
                       bn_gamma.reshape(1, D), bn_beta.reshape(1, D))
